# baseline (speedup 1.0000x reference)
import functools

import jax
import jax.numpy as jnp
from jax import lax
from jax.experimental import pallas as pl
from jax.experimental.pallas import tpu as pltpu
from jax.experimental.pallas import tpu_sc as plsc

B = 4096
BLOCK_R = 256
S_SC = 1024
NW = 32
RPW = S_SC // NW
_ROT_A = (13, 15, 26, 6)
_ROT_B = (17, 29, 16, 24)
_TINY = 1.1754943508222875e-38
_NEG_INF = float("-inf")


def _threefry_round(x0, x1, r):
    x0 = x0 + x1
    x1 = (x1 << r) | (x1 >> (32 - r))
    x1 = x0 ^ x1
    return x0, x1


def _threefry_bits(lo, ks1, ks2):
    k21 = ks2 + jnp.uint32(1)
    k13 = ks1 + jnp.uint32(3)
    k24 = ks2 + jnp.uint32(4)
    x1 = lo + ks1
    x0 = x1
    x1 = x1 ^ ((x1 << 13) | (x1 >> 19))
    for r in _ROT_A[1:]:
        x0, x1 = _threefry_round(x0, x1, r)
    x0 = x0 + ks1
    x1 = x1 + k21
    for r in _ROT_B:
        x0, x1 = _threefry_round(x0, x1, r)
    x0 = x0 + ks2
    x1 = x1 + jnp.uint32(2)
    for r in _ROT_A:
        x0, x1 = _threefry_round(x0, x1, r)
    x1 = x1 + k13
    for r in _ROT_B:
        x0, x1 = _threefry_round(x0, x1, r)
    x0 = x0 + ks1
    x1 = x1 + k24
    for r in _ROT_A:
        x0, x1 = _threefry_round(x0, x1, r)
    x0 = x0 + ks2
    x1 = x1 + jnp.uint32(5)
    return x0 ^ x1


def _gumbel(bits):
    fb = (bits >> 9) | jnp.uint32(0x3F800000)
    f = jax.lax.bitcast_convert_type(fb, jnp.float32) - jnp.float32(1.0)
    u = jnp.maximum(f, jnp.float32(_TINY))
    return -jnp.log(-jnp.log(u))


def _sample_value(key_vals, d, colid):
    m = jnp.max(key_vals, axis=1, keepdims=True)
    idx = jnp.min(jnp.where(key_vals == m, colid, jnp.int32(B)), axis=1,
                  keepdims=True)
    return jnp.max(jnp.where(colid == idx, d, jnp.float32(_NEG_INF)), axis=1,
                   keepdims=True)


def _select_keys(mask):
    ks1 = jnp.where(mask, jnp.uint32(123), jnp.uint32(456))
    return ks1, ks1 ^ jnp.uint32(0x1BD11BDA)


def _finish(d, g, mask, out_ref):
    key_pos = jnp.where(mask, d + g, jnp.float32(_NEG_INF))
    key_neg = jnp.where(mask, jnp.float32(_NEG_INF), g - d)
    colid = jax.lax.broadcasted_iota(jnp.int32, (BLOCK_R, B), 1)
    pos = _sample_value(key_pos, d, colid)
    neg = _sample_value(key_neg, d, colid)
    x = pos - neg
    out_ref[...] = jnp.maximum(x, 0.0) + jnp.log1p(jnp.exp(-jnp.abs(x)))


def _tc_main_body(dist_ref, prow_ref, pcol_ref, out_ref):
    i = pl.program_id(0)
    d = dist_ref[...]
    mask = pcol_ref[...] == prow_ref[...]
    row = jax.lax.broadcasted_iota(jnp.uint32, (BLOCK_R, B), 0)
    col = jax.lax.broadcasted_iota(jnp.uint32, (BLOCK_R, B), 1)
    lo = ((jnp.uint32(S_SC) + jnp.uint32(i * BLOCK_R) + row) << 12) | col
    ks1, ks2 = _select_keys(mask)
    g = _gumbel(_threefry_bits(lo, ks1, ks2))
    _finish(d, g, mask, out_ref)


def _tc_rest_body(dist_ref, bits_ref, prow_ref, pcol_ref, out_ref):
    d = dist_ref[...]
    mask = pcol_ref[...] == prow_ref[...]
    g = _gumbel(bits_ref[...])
    _finish(d, g, mask, out_ref)


_GROUP = 8
_UNROLL = 8


def _sc_bits_body(prow_hbm, pidsb_hbm, bits_hbm, prow_v, pidb_v, rowbuf):
    wid = lax.axis_index("s") * 2 + lax.axis_index("c")
    base = wid * RPW
    pltpu.sync_copy(prow_hbm, prow_v)
    pltpu.sync_copy(pidsb_hbm.at[pl.ds(base, RPW)], pidb_v)
    lane = lax.iota(jnp.uint32, 16)

    def group_body(gi, carry):
        gbase = base + gi * _GROUP
        for rr in range(_GROUP):
            r = gi * _GROUP + rr
            pv = pidb_v[r, :]
            robase = lax.convert_element_type((gbase + rr) * B, jnp.uint32)

            def chunk_body(cb, inner_carry, rr=rr, pv=pv, robase=robase):
                for u in range(_UNROLL):
                    c = cb * _UNROLL + u
                    pr = prow_v[pl.ds(c * 16, 16)]
                    ks1 = jnp.where(pr == pv, jnp.uint32(123), jnp.uint32(456))
                    ks2 = ks1 ^ jnp.uint32(0x1BD11BDA)
                    lo = lane + (robase
                                 + lax.convert_element_type(c * 16, jnp.uint32))
                    rowbuf[rr, pl.ds(c * 16, 16)] = _threefry_bits(lo, ks1, ks2)
                return inner_carry

            lax.fori_loop(0, B // (16 * _UNROLL), chunk_body, 0)
        pltpu.sync_copy(rowbuf, bits_hbm.at[pl.ds(gbase, _GROUP)])
        return carry

    lax.fori_loop(0, RPW // _GROUP, group_body, 0)


_sc_bits = pl.kernel(
    _sc_bits_body,
    out_type=jax.ShapeDtypeStruct((S_SC, B), jnp.uint32),
    mesh=plsc.VectorSubcoreMesh(core_axis_name="c", subcore_axis_name="s"),
    scratch_types=[
        pltpu.VMEM((B,), jnp.int32),
        pltpu.VMEM((RPW, 16), jnp.int32),
        pltpu.VMEM((_GROUP, B), jnp.uint32),
    ],
)


@jax.jit
def kernel(dist, pids):
    prow = pids.reshape(1, B)
    pcol = pids.reshape(B, 1)
    pidsb = jnp.broadcast_to(pids[:S_SC, None], (S_SC, 16))
    bits = _sc_bits(pids, pidsb)
    out_main = pl.pallas_call(
        _tc_main_body,
        grid=((B - S_SC) // BLOCK_R,),
        in_specs=[
            pl.BlockSpec((BLOCK_R, B), lambda i: (i + S_SC // BLOCK_R, 0)),
            pl.BlockSpec((1, B), lambda i: (0, 0)),
            pl.BlockSpec((BLOCK_R, 1), lambda i: (i + S_SC // BLOCK_R, 0)),
        ],
        out_specs=pl.BlockSpec((BLOCK_R, 1), lambda i: (i, 0)),
        out_shape=jax.ShapeDtypeStruct((B - S_SC, 1), jnp.float32),
    )(dist, prow, pcol)
    out_rest = pl.pallas_call(
        _tc_rest_body,
        grid=(S_SC // BLOCK_R,),
        in_specs=[
            pl.BlockSpec((BLOCK_R, B), lambda i: (i, 0)),
            pl.BlockSpec((BLOCK_R, B), lambda i: (i, 0)),
            pl.BlockSpec((1, B), lambda i: (0, 0)),
            pl.BlockSpec((BLOCK_R, 1), lambda i: (i, 0)),
        ],
        out_specs=pl.BlockSpec((BLOCK_R, 1), lambda i: (i, 0)),
        out_shape=jax.ShapeDtypeStruct((S_SC, 1), jnp.float32),
    )(dist, bits, prow, pcol)
    return jnp.concatenate([out_rest.reshape(S_SC), out_main.reshape(B - S_SC)])


# speedup vs baseline: 1.2560x; 1.2560x over previous
import functools

import jax
import jax.numpy as jnp
from jax import lax
from jax.experimental import pallas as pl
from jax.experimental.pallas import tpu as pltpu
from jax.experimental.pallas import tpu_sc as plsc

B = 4096
BLOCK_R = 256
S_SC = 1024
NW = 32
RPW = S_SC // NW
_ROT_A = (13, 15, 26, 6)
_ROT_B = (17, 29, 16, 24)
_TINY = 1.1754943508222875e-38
_NEG_INF = float("-inf")


def _threefry_round(x0, x1, r):
    x0 = x0 + x1
    x1 = (x1 << r) | (x1 >> (32 - r))
    x1 = x0 ^ x1
    return x0, x1


def _threefry_bits(lo, ks1, ks2):
    k21 = ks2 + jnp.uint32(1)
    k13 = ks1 + jnp.uint32(3)
    k24 = ks2 + jnp.uint32(4)
    x1 = lo + ks1
    x0 = x1
    x1 = x1 ^ ((x1 << 13) | (x1 >> 19))
    for r in _ROT_A[1:]:
        x0, x1 = _threefry_round(x0, x1, r)
    x0 = x0 + ks1
    x1 = x1 + k21
    for r in _ROT_B:
        x0, x1 = _threefry_round(x0, x1, r)
    x0 = x0 + ks2
    x1 = x1 + jnp.uint32(2)
    for r in _ROT_A:
        x0, x1 = _threefry_round(x0, x1, r)
    x1 = x1 + k13
    for r in _ROT_B:
        x0, x1 = _threefry_round(x0, x1, r)
    x0 = x0 + ks1
    x1 = x1 + k24
    for r in _ROT_A:
        x0, x1 = _threefry_round(x0, x1, r)
    x0 = x0 + ks2
    x1 = x1 + jnp.uint32(5)
    return x0 ^ x1


def _gumbel(bits):
    fb = (bits >> 9) | jnp.uint32(0x3F800000)
    f = jax.lax.bitcast_convert_type(fb, jnp.float32) - jnp.float32(1.0)
    u = jnp.maximum(f, jnp.float32(_TINY))
    return -jnp.log(-jnp.log(u))


def _sample_value(key_vals, d, colid):
    m = jnp.max(key_vals, axis=1, keepdims=True)
    idx = jnp.min(jnp.where(key_vals == m, colid, jnp.int32(B)), axis=1,
                  keepdims=True)
    return jnp.max(jnp.where(colid == idx, d, jnp.float32(_NEG_INF)), axis=1,
                   keepdims=True)


def _select_keys(mask):
    ks1 = jnp.where(mask, jnp.uint32(123), jnp.uint32(456))
    return ks1, ks1 ^ jnp.uint32(0x1BD11BDA)


def _finish(d, g, mask, out_ref):
    key_pos = jnp.where(mask, d + g, jnp.float32(_NEG_INF))
    key_neg = jnp.where(mask, jnp.float32(_NEG_INF), g - d)
    colid = jax.lax.broadcasted_iota(jnp.int32, (BLOCK_R, B), 1)
    pos = _sample_value(key_pos, d, colid)
    neg = _sample_value(key_neg, d, colid)
    x = pos - neg
    out_ref[...] = jnp.maximum(x, 0.0) + jnp.log1p(jnp.exp(-jnp.abs(x)))


def _tc_main_body(dist_ref, prow_ref, pcol_ref, out_ref):
    i = pl.program_id(0)
    d = dist_ref[...]
    mask = pcol_ref[...] == prow_ref[...]
    row = jax.lax.broadcasted_iota(jnp.uint32, (BLOCK_R, B), 0)
    col = jax.lax.broadcasted_iota(jnp.uint32, (BLOCK_R, B), 1)
    lo = ((jnp.uint32(S_SC) + jnp.uint32(i * BLOCK_R) + row) << 12) | col
    ks1, ks2 = _select_keys(mask)
    g = _gumbel(_threefry_bits(lo, ks1, ks2))
    _finish(d, g, mask, out_ref)


def _tc_rest_body(dist_ref, bits_ref, prow_ref, pcol_ref, out_ref):
    d = dist_ref[...]
    mask = pcol_ref[...] == prow_ref[...]
    g = _gumbel(bits_ref[...])
    _finish(d, g, mask, out_ref)


_GROUP = 8
_UNROLL = 8


def _sc_bits_body(prow_hbm, pidsb_hbm, bits_hbm, prow_v, pidb_v, rowbuf):
    wid = lax.axis_index("s") * 2 + lax.axis_index("c")
    base = wid * RPW
    pltpu.sync_copy(prow_hbm, prow_v)
    pltpu.sync_copy(pidsb_hbm.at[pl.ds(base, RPW)], pidb_v)
    lane = lax.iota(jnp.uint32, 16)

    def group_body(gi, carry):
        gbase = base + gi * _GROUP
        for rr in range(_GROUP):
            r = gi * _GROUP + rr
            pv = pidb_v[r, :]
            robase = lax.convert_element_type((gbase + rr) * B, jnp.uint32)

            @plsc.parallel_loop(0, B // 16, 1, unroll=_UNROLL)
            def chunk_body(c, rr=rr, pv=pv, robase=robase):
                pr = prow_v[pl.ds(c * 16, 16)]
                ks1 = jnp.where(pr == pv, jnp.uint32(123), jnp.uint32(456))
                ks2 = ks1 ^ jnp.uint32(0x1BD11BDA)
                lo = lane + (robase
                             + lax.convert_element_type(c * 16, jnp.uint32))
                rowbuf[rr, pl.ds(c * 16, 16)] = _threefry_bits(lo, ks1, ks2)
        pltpu.sync_copy(rowbuf, bits_hbm.at[pl.ds(gbase, _GROUP)])
        return carry

    lax.fori_loop(0, RPW // _GROUP, group_body, 0)


_sc_bits = pl.kernel(
    _sc_bits_body,
    out_type=jax.ShapeDtypeStruct((S_SC, B), jnp.uint32),
    mesh=plsc.VectorSubcoreMesh(core_axis_name="c", subcore_axis_name="s"),
    scratch_types=[
        pltpu.VMEM((B,), jnp.int32),
        pltpu.VMEM((RPW, 16), jnp.int32),
        pltpu.VMEM((_GROUP, B), jnp.uint32),
    ],
)


@jax.jit
def kernel(dist, pids):
    prow = pids.reshape(1, B)
    pcol = pids.reshape(B, 1)
    pidsb = jnp.broadcast_to(pids[:S_SC, None], (S_SC, 16))
    bits = _sc_bits(pids, pidsb)
    out_main = pl.pallas_call(
        _tc_main_body,
        grid=((B - S_SC) // BLOCK_R,),
        in_specs=[
            pl.BlockSpec((BLOCK_R, B), lambda i: (i + S_SC // BLOCK_R, 0)),
            pl.BlockSpec((1, B), lambda i: (0, 0)),
            pl.BlockSpec((BLOCK_R, 1), lambda i: (i + S_SC // BLOCK_R, 0)),
        ],
        out_specs=pl.BlockSpec((BLOCK_R, 1), lambda i: (i, 0)),
        out_shape=jax.ShapeDtypeStruct((B - S_SC, 1), jnp.float32),
    )(dist, prow, pcol)
    out_rest = pl.pallas_call(
        _tc_rest_body,
        grid=(S_SC // BLOCK_R,),
        in_specs=[
            pl.BlockSpec((BLOCK_R, B), lambda i: (i, 0)),
            pl.BlockSpec((BLOCK_R, B), lambda i: (i, 0)),
            pl.BlockSpec((1, B), lambda i: (0, 0)),
            pl.BlockSpec((BLOCK_R, 1), lambda i: (i, 0)),
        ],
        out_specs=pl.BlockSpec((BLOCK_R, 1), lambda i: (i, 0)),
        out_shape=jax.ShapeDtypeStruct((S_SC, 1), jnp.float32),
    )(dist, bits, prow, pcol)
    return jnp.concatenate([out_rest.reshape(S_SC), out_main.reshape(B - S_SC)])


# speedup vs baseline: 1.3321x; 1.0606x over previous
import functools

import jax
import jax.numpy as jnp
from jax import lax
from jax.experimental import pallas as pl
from jax.experimental.pallas import tpu as pltpu
from jax.experimental.pallas import tpu_sc as plsc

B = 4096
BLOCK_R = 256
S_SC = 1280
NW = 32
RPW = S_SC // NW
_ROT_A = (13, 15, 26, 6)
_ROT_B = (17, 29, 16, 24)
_TINY = 1.1754943508222875e-38
_NEG_INF = float("-inf")


def _threefry_round(x0, x1, r):
    x0 = x0 + x1
    x1 = (x1 << r) | (x1 >> (32 - r))
    x1 = x0 ^ x1
    return x0, x1


def _threefry_bits(lo, ks1, ks2):
    k21 = ks2 + jnp.uint32(1)
    k13 = ks1 + jnp.uint32(3)
    k24 = ks2 + jnp.uint32(4)
    x1 = lo + ks1
    x0 = x1
    x1 = x1 ^ ((x1 << 13) | (x1 >> 19))
    for r in _ROT_A[1:]:
        x0, x1 = _threefry_round(x0, x1, r)
    x0 = x0 + ks1
    x1 = x1 + k21
    for r in _ROT_B:
        x0, x1 = _threefry_round(x0, x1, r)
    x0 = x0 + ks2
    x1 = x1 + jnp.uint32(2)
    for r in _ROT_A:
        x0, x1 = _threefry_round(x0, x1, r)
    x1 = x1 + k13
    for r in _ROT_B:
        x0, x1 = _threefry_round(x0, x1, r)
    x0 = x0 + ks1
    x1 = x1 + k24
    for r in _ROT_A:
        x0, x1 = _threefry_round(x0, x1, r)
    x0 = x0 + ks2
    x1 = x1 + jnp.uint32(5)
    return x0 ^ x1


def _gumbel(bits):
    fb = (bits >> 9) | jnp.uint32(0x3F800000)
    f = jax.lax.bitcast_convert_type(fb, jnp.float32) - jnp.float32(1.0)
    u = jnp.maximum(f, jnp.float32(_TINY))
    return -jnp.log(-jnp.log(u))


def _sample_value(key_vals, d, colid):
    m = jnp.max(key_vals, axis=1, keepdims=True)
    idx = jnp.min(jnp.where(key_vals == m, colid, jnp.int32(B)), axis=1,
                  keepdims=True)
    return jnp.max(jnp.where(colid == idx, d, jnp.float32(_NEG_INF)), axis=1,
                   keepdims=True)


def _select_keys(mask):
    ks1 = jnp.where(mask, jnp.uint32(123), jnp.uint32(456))
    return ks1, ks1 ^ jnp.uint32(0x1BD11BDA)


def _finish(d, g, mask, out_ref):
    key_pos = jnp.where(mask, d + g, jnp.float32(_NEG_INF))
    key_neg = jnp.where(mask, jnp.float32(_NEG_INF), g - d)
    colid = jax.lax.broadcasted_iota(jnp.int32, (BLOCK_R, B), 1)
    pos = _sample_value(key_pos, d, colid)
    neg = _sample_value(key_neg, d, colid)
    x = pos - neg
    out_ref[...] = jnp.maximum(x, 0.0) + jnp.log1p(jnp.exp(-jnp.abs(x)))


def _tc_main_body(dist_ref, prow_ref, pcol_ref, out_ref):
    i = pl.program_id(0)
    d = dist_ref[...]
    mask = pcol_ref[...] == prow_ref[...]
    row = jax.lax.broadcasted_iota(jnp.uint32, (BLOCK_R, B), 0)
    col = jax.lax.broadcasted_iota(jnp.uint32, (BLOCK_R, B), 1)
    lo = ((jnp.uint32(S_SC) + jnp.uint32(i * BLOCK_R) + row) << 12) | col
    ks1, ks2 = _select_keys(mask)
    g = _gumbel(_threefry_bits(lo, ks1, ks2))
    _finish(d, g, mask, out_ref)


def _tc_rest_body(dist_ref, bits_ref, prow_ref, pcol_ref, out_ref):
    d = dist_ref[...]
    mask = pcol_ref[...] == prow_ref[...]
    g = _gumbel(bits_ref[...])
    _finish(d, g, mask, out_ref)


_GROUP = 8
_UNROLL = 8


def _sc_bits_body(prow_hbm, pidsb_hbm, bits_hbm, prow_v, pidb_v, rowbuf):
    wid = lax.axis_index("s") * 2 + lax.axis_index("c")
    base = wid * RPW
    pltpu.sync_copy(prow_hbm, prow_v)
    pltpu.sync_copy(pidsb_hbm.at[pl.ds(base, RPW)], pidb_v)
    lane = lax.iota(jnp.uint32, 16)

    def group_body(gi, carry):
        gbase = base + gi * _GROUP
        for rr in range(_GROUP):
            r = gi * _GROUP + rr
            pv = pidb_v[r, :]
            robase = lax.convert_element_type((gbase + rr) * B, jnp.uint32)

            @plsc.parallel_loop(0, B // 16, 1, unroll=_UNROLL)
            def chunk_body(c, rr=rr, pv=pv, robase=robase):
                pr = prow_v[pl.ds(c * 16, 16)]
                ks1 = jnp.where(pr == pv, jnp.uint32(123), jnp.uint32(456))
                ks2 = ks1 ^ jnp.uint32(0x1BD11BDA)
                lo = lane + (robase
                             + lax.convert_element_type(c * 16, jnp.uint32))
                rowbuf[rr, pl.ds(c * 16, 16)] = _threefry_bits(lo, ks1, ks2)
        pltpu.sync_copy(rowbuf, bits_hbm.at[pl.ds(gbase, _GROUP)])
        return carry

    lax.fori_loop(0, RPW // _GROUP, group_body, 0)


_sc_bits = pl.kernel(
    _sc_bits_body,
    out_type=jax.ShapeDtypeStruct((S_SC, B), jnp.uint32),
    mesh=plsc.VectorSubcoreMesh(core_axis_name="c", subcore_axis_name="s"),
    scratch_types=[
        pltpu.VMEM((B,), jnp.int32),
        pltpu.VMEM((RPW, 16), jnp.int32),
        pltpu.VMEM((_GROUP, B), jnp.uint32),
    ],
)


@jax.jit
def kernel(dist, pids):
    prow = pids.reshape(1, B)
    pcol = pids.reshape(B, 1)
    pidsb = jnp.broadcast_to(pids[:S_SC, None], (S_SC, 16))
    bits = _sc_bits(pids, pidsb)
    out_main = pl.pallas_call(
        _tc_main_body,
        grid=((B - S_SC) // BLOCK_R,),
        in_specs=[
            pl.BlockSpec((BLOCK_R, B), lambda i: (i + S_SC // BLOCK_R, 0)),
            pl.BlockSpec((1, B), lambda i: (0, 0)),
            pl.BlockSpec((BLOCK_R, 1), lambda i: (i + S_SC // BLOCK_R, 0)),
        ],
        out_specs=pl.BlockSpec((BLOCK_R, 1), lambda i: (i, 0)),
        out_shape=jax.ShapeDtypeStruct((B - S_SC, 1), jnp.float32),
    )(dist, prow, pcol)
    out_rest = pl.pallas_call(
        _tc_rest_body,
        grid=(S_SC // BLOCK_R,),
        in_specs=[
            pl.BlockSpec((BLOCK_R, B), lambda i: (i, 0)),
            pl.BlockSpec((BLOCK_R, B), lambda i: (i, 0)),
            pl.BlockSpec((1, B), lambda i: (0, 0)),
            pl.BlockSpec((BLOCK_R, 1), lambda i: (i, 0)),
        ],
        out_specs=pl.BlockSpec((BLOCK_R, 1), lambda i: (i, 0)),
        out_shape=jax.ShapeDtypeStruct((S_SC, 1), jnp.float32),
    )(dist, bits, prow, pcol)
    return jnp.concatenate([out_rest.reshape(S_SC), out_main.reshape(B - S_SC)])


# speedup vs baseline: 1.3323x; 1.0002x over previous
import functools

import jax
import jax.numpy as jnp
from jax import lax
from jax.experimental import pallas as pl
from jax.experimental.pallas import tpu as pltpu
from jax.experimental.pallas import tpu_sc as plsc

B = 4096
BLOCK_R = 256
S_SC = 1280
NW = 32
RPW = S_SC // NW
_ROT_A = (13, 15, 26, 6)
_ROT_B = (17, 29, 16, 24)
_TINY = 1.1754943508222875e-38
_NEG_INF = float("-inf")


def _threefry_round(x0, x1, r):
    x0 = x0 + x1
    x1 = (x1 << r) | (x1 >> (32 - r))
    x1 = x0 ^ x1
    return x0, x1


def _threefry_bits(lo, ks1, ks2):
    k21 = ks2 + jnp.uint32(1)
    k13 = ks1 + jnp.uint32(3)
    k24 = ks2 + jnp.uint32(4)
    x1 = lo + ks1
    x0 = x1
    x1 = x1 ^ ((x1 << 13) | (x1 >> 19))
    for r in _ROT_A[1:]:
        x0, x1 = _threefry_round(x0, x1, r)
    x0 = x0 + ks1
    x1 = x1 + k21
    for r in _ROT_B:
        x0, x1 = _threefry_round(x0, x1, r)
    x0 = x0 + ks2
    x1 = x1 + jnp.uint32(2)
    for r in _ROT_A:
        x0, x1 = _threefry_round(x0, x1, r)
    x1 = x1 + k13
    for r in _ROT_B:
        x0, x1 = _threefry_round(x0, x1, r)
    x0 = x0 + ks1
    x1 = x1 + k24
    for r in _ROT_A:
        x0, x1 = _threefry_round(x0, x1, r)
    x0 = x0 + ks2
    x1 = x1 + jnp.uint32(5)
    return x0 ^ x1


def _gumbel(bits):
    fb = (bits >> 9) | jnp.uint32(0x3F800000)
    f = jax.lax.bitcast_convert_type(fb, jnp.float32) - jnp.float32(1.0)
    u = jnp.maximum(f, jnp.float32(_TINY))
    return -jnp.log(-jnp.log(u))


def _sample_value(key_vals, d, colid):
    m = jnp.max(key_vals, axis=1, keepdims=True)
    idx = jnp.min(jnp.where(key_vals == m, colid, jnp.int32(B)), axis=1,
                  keepdims=True)
    return jnp.max(jnp.where(colid == idx, d, jnp.float32(_NEG_INF)), axis=1,
                   keepdims=True)


def _select_keys(mask):
    ks1 = jnp.where(mask, jnp.uint32(123), jnp.uint32(456))
    return ks1, ks1 ^ jnp.uint32(0x1BD11BDA)


def _finish(d, g, mask, out_ref):
    key_pos = jnp.where(mask, d + g, jnp.float32(_NEG_INF))
    key_neg = jnp.where(mask, jnp.float32(_NEG_INF), g - d)
    colid = jax.lax.broadcasted_iota(jnp.int32, (BLOCK_R, B), 1)
    pos = _sample_value(key_pos, d, colid)
    neg = _sample_value(key_neg, d, colid)
    x = pos - neg
    out_ref[...] = jnp.maximum(x, 0.0) + jnp.log1p(jnp.exp(-jnp.abs(x)))


def _tc_main_body(dist_ref, prow_ref, pcol_ref, out_ref):
    i = pl.program_id(0)
    d = dist_ref[...]
    mask = pcol_ref[...] == prow_ref[...]
    row = jax.lax.broadcasted_iota(jnp.uint32, (BLOCK_R, B), 0)
    col = jax.lax.broadcasted_iota(jnp.uint32, (BLOCK_R, B), 1)
    lo = ((jnp.uint32(S_SC) + jnp.uint32(i * BLOCK_R) + row) << 12) | col
    ks1, ks2 = _select_keys(mask)
    g = _gumbel(_threefry_bits(lo, ks1, ks2))
    _finish(d, g, mask, out_ref)


def _tc_rest_body(dist_ref, bits_ref, prow_ref, pcol_ref, out_ref):
    d = dist_ref[...]
    mask = pcol_ref[...] == prow_ref[...]
    g = _gumbel(bits_ref[...])
    _finish(d, g, mask, out_ref)


_GROUP = 8
_UNROLL = 16


def _sc_bits_body(prow_hbm, pidsb_hbm, bits_hbm, prow_v, pidb_v, rowbuf):
    wid = lax.axis_index("s") * 2 + lax.axis_index("c")
    base = wid * RPW
    pltpu.sync_copy(prow_hbm, prow_v)
    pltpu.sync_copy(pidsb_hbm.at[pl.ds(base, RPW)], pidb_v)
    lane = lax.iota(jnp.uint32, 16)

    def group_body(gi, carry):
        gbase = base + gi * _GROUP
        for rr in range(_GROUP):
            r = gi * _GROUP + rr
            pv = pidb_v[r, :]
            robase = lax.convert_element_type((gbase + rr) * B, jnp.uint32)

            @plsc.parallel_loop(0, B // 16, 1, unroll=_UNROLL)
            def chunk_body(c, rr=rr, pv=pv, robase=robase):
                pr = prow_v[pl.ds(c * 16, 16)]
                ks1 = jnp.where(pr == pv, jnp.uint32(123), jnp.uint32(456))
                ks2 = ks1 ^ jnp.uint32(0x1BD11BDA)
                lo = lane + (robase
                             + lax.convert_element_type(c * 16, jnp.uint32))
                rowbuf[rr, pl.ds(c * 16, 16)] = _threefry_bits(lo, ks1, ks2)
        pltpu.sync_copy(rowbuf, bits_hbm.at[pl.ds(gbase, _GROUP)])
        return carry

    lax.fori_loop(0, RPW // _GROUP, group_body, 0)


_sc_bits = pl.kernel(
    _sc_bits_body,
    out_type=jax.ShapeDtypeStruct((S_SC, B), jnp.uint32),
    mesh=plsc.VectorSubcoreMesh(core_axis_name="c", subcore_axis_name="s"),
    scratch_types=[
        pltpu.VMEM((B,), jnp.int32),
        pltpu.VMEM((RPW, 16), jnp.int32),
        pltpu.VMEM((_GROUP, B), jnp.uint32),
    ],
)


@jax.jit
def kernel(dist, pids):
    prow = pids.reshape(1, B)
    pcol = pids.reshape(B, 1)
    pidsb = jnp.broadcast_to(pids[:S_SC, None], (S_SC, 16))
    bits = _sc_bits(pids, pidsb)
    out_main = pl.pallas_call(
        _tc_main_body,
        grid=((B - S_SC) // BLOCK_R,),
        in_specs=[
            pl.BlockSpec((BLOCK_R, B), lambda i: (i + S_SC // BLOCK_R, 0)),
            pl.BlockSpec((1, B), lambda i: (0, 0)),
            pl.BlockSpec((BLOCK_R, 1), lambda i: (i + S_SC // BLOCK_R, 0)),
        ],
        out_specs=pl.BlockSpec((BLOCK_R, 1), lambda i: (i, 0)),
        out_shape=jax.ShapeDtypeStruct((B - S_SC, 1), jnp.float32),
    )(dist, prow, pcol)
    out_rest = pl.pallas_call(
        _tc_rest_body,
        grid=(S_SC // BLOCK_R,),
        in_specs=[
            pl.BlockSpec((BLOCK_R, B), lambda i: (i, 0)),
            pl.BlockSpec((BLOCK_R, B), lambda i: (i, 0)),
            pl.BlockSpec((1, B), lambda i: (0, 0)),
            pl.BlockSpec((BLOCK_R, 1), lambda i: (i, 0)),
        ],
        out_specs=pl.BlockSpec((BLOCK_R, 1), lambda i: (i, 0)),
        out_shape=jax.ShapeDtypeStruct((S_SC, 1), jnp.float32),
    )(dist, bits, prow, pcol)
    return jnp.concatenate([out_rest.reshape(S_SC), out_main.reshape(B - S_SC)])
